# baseline (speedup 1.0000x reference)
import functools

import jax
import jax.numpy as jnp
from jax import lax
from jax.experimental import pallas as pl
from jax.experimental.pallas import tpu as pltpu
from jax.experimental.pallas import tpu_sc as plsc

_TOTAL = 4 * 4096 * 2048
_GRP = 2048
_GR = 8
_COLS = 2048
_NW = 32
_G_W = _GRP // _NW
_UNROLL = 8


def _sc_loss_partials(o3, t3):
    mesh = plsc.VectorSubcoreMesh(core_axis_name="c", subcore_axis_name="s")

    @functools.partial(
        pl.kernel,
        mesh=mesh,
        out_type=jax.ShapeDtypeStruct((_NW, 16), jnp.float32),
        scratch_types=[
            pltpu.VMEM((2, _GR, _COLS), jnp.float32),
            pltpu.VMEM((2, _GR, _COLS), jnp.float32),
            pltpu.VMEM((16,), jnp.float32),
            pltpu.SemaphoreType.DMA,
            pltpu.SemaphoreType.DMA,
            pltpu.SemaphoreType.DMA,
            pltpu.SemaphoreType.DMA,
        ],
    )
    def k(o_hbm, t_hbm, out_hbm, o_buf, t_buf, acc_vm, so0, so1, st0, st1):
        wid = lax.axis_index("s") * 2 + lax.axis_index("c")
        g0 = wid * _G_W
        sems_o = (so0, so1)
        sems_t = (st0, st1)

        def copy_o(k_idx, b):
            return pltpu.make_async_copy(
                o_hbm.at[g0 + k_idx], o_buf.at[b], sems_o[b])

        def copy_t(k_idx, b):
            return pltpu.make_async_copy(
                t_hbm.at[g0 + k_idx], t_buf.at[b], sems_t[b])

        def start(k_idx, b):
            copy_o(k_idx, b).start()
            copy_t(k_idx, b).start()

        def wait(k_idx, b):
            copy_o(k_idx, b).wait()
            copy_t(k_idx, b).wait()

        def chunk_sum(b, accs):
            def vbody(v, a):
                out = list(a)
                for u in range(_UNROLL):
                    off = v * _UNROLL * 16 + u * 16
                    for r in range(_GR):
                        o = o_buf.at[b].at[r][pl.ds(off, 16)]
                        t = t_buf.at[b].at[r][pl.ds(off, 16)]
                        d = jnp.where(t != 0.0, o - t, 0.0)
                        out[u] = out[u] + d * d
                return tuple(out)

            return lax.fori_loop(0, _COLS // (16 * _UNROLL), vbody, accs)

        start(0, 0)
        start(1, 1)

        def gbody(gg, accs):
            for b in (0, 1):
                k_idx = 2 * gg + b
                wait(k_idx, b)
                accs = chunk_sum(b, accs)
                start(k_idx + 2, b)
            return accs

        zero = jnp.zeros((16,), jnp.float32)
        accs = lax.fori_loop(0, (_G_W - 2) // 2, gbody, (zero,) * _UNROLL)
        for b in (0, 1):
            wait(_G_W - 2 + b, b)
            accs = chunk_sum(b, accs)

        acc = accs[0]
        for u in range(1, _UNROLL):
            acc = acc + accs[u]
        acc_vm[...] = acc
        pltpu.sync_copy(acc_vm, out_hbm.at[wid])

    return k(o3, t3)


def kernel(output, target):
    o3 = output.reshape(_GRP, _GR, _COLS)
    t3 = target.reshape(_GRP, _GR, _COLS)
    partials = _sc_loss_partials(o3, t3)
    return jnp.sum(partials) / _TOTAL


# speedup vs baseline: 5.2032x; 5.2032x over previous
import functools

import jax
import jax.numpy as jnp
from jax import lax
from jax.experimental import pallas as pl
from jax.experimental.pallas import tpu as pltpu
from jax.experimental.pallas import tpu_sc as plsc

_TOTAL = 4 * 4096 * 2048
_GRP = 2048
_GR = 8
_COLS = 2048
_NW = 32
_G_W = _GRP // _NW
_UNROLL = 8


def _sc_loss_partials(o3, t3):
    mesh = plsc.VectorSubcoreMesh(core_axis_name="c", subcore_axis_name="s")

    @functools.partial(
        pl.kernel,
        mesh=mesh,
        out_type=jax.ShapeDtypeStruct((_NW, 16), jnp.float32),
        scratch_types=[
            pltpu.VMEM((2, _GR, _COLS), jnp.float32),
            pltpu.VMEM((2, _GR, _COLS), jnp.float32),
            pltpu.VMEM((16,), jnp.float32),
            pltpu.SemaphoreType.DMA,
            pltpu.SemaphoreType.DMA,
            pltpu.SemaphoreType.DMA,
            pltpu.SemaphoreType.DMA,
        ],
    )
    def k(o_hbm, t_hbm, out_hbm, o_buf, t_buf, acc_vm, so0, so1, st0, st1):
        wid = lax.axis_index("s") * 2 + lax.axis_index("c")
        g0 = wid * _G_W
        sems_o = (so0, so1)
        sems_t = (st0, st1)

        def copy_o(k_idx, b):
            return pltpu.make_async_copy(
                o_hbm.at[g0 + k_idx], o_buf.at[b], sems_o[b])

        def copy_t(k_idx, b):
            return pltpu.make_async_copy(
                t_hbm.at[g0 + k_idx], t_buf.at[b], sems_t[b])

        def start(k_idx, b):
            copy_o(k_idx, b).start()
            copy_t(k_idx, b).start()

        def wait(k_idx, b):
            copy_o(k_idx, b).wait()
            copy_t(k_idx, b).wait()

        def chunk_sum(b, accs):
            def rbody(r, a_r):
                orow = o_buf.at[b].at[r]
                trow = t_buf.at[b].at[r]

                def vbody(v, a):
                    out = []
                    for u in range(_UNROLL):
                        off = v * _UNROLL * 16 + u * 16
                        o = orow[pl.ds(off, 16)]
                        t = trow[pl.ds(off, 16)]
                        d = jnp.where(jnp.abs(t) > 0.0, o - t, 0.0)
                        out.append(a[u] + d * d)
                    return tuple(out)

                return lax.fori_loop(0, _COLS // (16 * _UNROLL), vbody, a_r)

            return lax.fori_loop(0, _GR, rbody, accs)

        start(0, 0)
        start(1, 1)

        def gbody(gg, accs):
            for b in (0, 1):
                k_idx = 2 * gg + b
                wait(k_idx, b)
                accs = chunk_sum(b, accs)
                start(k_idx + 2, b)
            return accs

        zero = jnp.zeros((16,), jnp.float32)
        accs = lax.fori_loop(0, (_G_W - 2) // 2, gbody, (zero,) * _UNROLL)
        for b in (0, 1):
            wait(_G_W - 2 + b, b)
            accs = chunk_sum(b, accs)

        acc = accs[0]
        for u in range(1, _UNROLL):
            acc = acc + accs[u]
        acc_vm[...] = acc
        pltpu.sync_copy(acc_vm, out_hbm.at[wid])

    return k(o3, t3)


def kernel(output, target):
    o3 = output.reshape(_GRP, _GR, _COLS)
    t3 = target.reshape(_GRP, _GR, _COLS)
    partials = _sc_loss_partials(o3, t3)
    return jnp.sum(partials) / _TOTAL


# speedup vs baseline: 6.8820x; 1.3227x over previous
import functools

import jax
import jax.numpy as jnp
from jax import lax
from jax.experimental import pallas as pl
from jax.experimental.pallas import tpu as pltpu
from jax.experimental.pallas import tpu_sc as plsc

_TOTAL = 4 * 4096 * 2048
_GRP = 2048
_GR = 8
_COLS = 2048
_NW = 32
_UNROLL = 8

_G_SC = 768
_G_W = _G_SC // _NW

_TC_ROW0 = _G_SC * _GR
_TC_BLOCK = 512
_TC_GRID = (_GRP - _G_SC) * _GR // _TC_BLOCK


def _sc_loss_partials(o3, t3):
    mesh = plsc.VectorSubcoreMesh(core_axis_name="c", subcore_axis_name="s")

    @functools.partial(
        pl.kernel,
        mesh=mesh,
        out_type=jax.ShapeDtypeStruct((_NW, 16), jnp.float32),
        scratch_types=[
            pltpu.VMEM((2, _GR, _COLS), jnp.float32),
            pltpu.VMEM((2, _GR, _COLS), jnp.float32),
            pltpu.VMEM((16,), jnp.float32),
            pltpu.SemaphoreType.DMA,
            pltpu.SemaphoreType.DMA,
            pltpu.SemaphoreType.DMA,
            pltpu.SemaphoreType.DMA,
        ],
    )
    def k(o_hbm, t_hbm, out_hbm, o_buf, t_buf, acc_vm, so0, so1, st0, st1):
        wid = lax.axis_index("s") * 2 + lax.axis_index("c")
        g0 = wid * _G_W
        sems_o = (so0, so1)
        sems_t = (st0, st1)

        def copy_o(k_idx, b):
            return pltpu.make_async_copy(
                o_hbm.at[g0 + k_idx], o_buf.at[b], sems_o[b])

        def copy_t(k_idx, b):
            return pltpu.make_async_copy(
                t_hbm.at[g0 + k_idx], t_buf.at[b], sems_t[b])

        def start(k_idx, b):
            copy_o(k_idx, b).start()
            copy_t(k_idx, b).start()

        def wait(k_idx, b):
            copy_o(k_idx, b).wait()
            copy_t(k_idx, b).wait()

        def chunk_sum(b, accs):
            def rbody(r, a_r):
                orow = o_buf.at[b].at[r]
                trow = t_buf.at[b].at[r]

                def vbody(v, a):
                    out = []
                    for u in range(_UNROLL):
                        off = v * _UNROLL * 16 + u * 16
                        o = orow[pl.ds(off, 16)]
                        t = trow[pl.ds(off, 16)]
                        d = jnp.where(jnp.abs(t) > 0.0, o - t, 0.0)
                        out.append(a[u] + d * d)
                    return tuple(out)

                return lax.fori_loop(0, _COLS // (16 * _UNROLL), vbody, a_r)

            return lax.fori_loop(0, _GR, rbody, accs)

        start(0, 0)
        start(1, 1)

        def gbody(gg, accs):
            for b in (0, 1):
                k_idx = 2 * gg + b
                wait(k_idx, b)
                accs = chunk_sum(b, accs)
                start(k_idx + 2, b)
            return accs

        zero = jnp.zeros((16,), jnp.float32)
        accs = lax.fori_loop(0, (_G_W - 2) // 2, gbody, (zero,) * _UNROLL)
        for b in (0, 1):
            wait(_G_W - 2 + b, b)
            accs = chunk_sum(b, accs)

        acc = accs[0]
        for u in range(1, _UNROLL):
            acc = acc + accs[u]
        acc_vm[...] = acc
        pltpu.sync_copy(acc_vm, out_hbm.at[wid])

    return k(o3, t3)


def _tc_body(o_ref, t_ref, out_ref):
    o = o_ref[...]
    t = t_ref[...]
    d = o - t
    sq = jnp.where(jnp.abs(t) > 0.0, d * d, 0.0)
    part = jnp.sum(sq)

    @pl.when(pl.program_id(0) == 0)
    def _():
        out_ref[0, 0] = 0.0

    out_ref[0, 0] += part


def _tc_loss_partial(o2, t2):
    row_blk0 = _TC_ROW0 // _TC_BLOCK
    total = pl.pallas_call(
        _tc_body,
        grid=(_TC_GRID,),
        in_specs=[
            pl.BlockSpec((_TC_BLOCK, _COLS), lambda i: (row_blk0 + i, 0)),
            pl.BlockSpec((_TC_BLOCK, _COLS), lambda i: (row_blk0 + i, 0)),
        ],
        out_specs=pl.BlockSpec(memory_space=pltpu.SMEM),
        out_shape=jax.ShapeDtypeStruct((1, 1), jnp.float32),
    )(o2, t2)
    return total[0, 0]


def kernel(output, target):
    o3 = output.reshape(_GRP, _GR, _COLS)
    t3 = target.reshape(_GRP, _GR, _COLS)
    o2 = output.reshape(_GRP * _GR, _COLS)
    t2 = target.reshape(_GRP * _GR, _COLS)
    sc_partials = _sc_loss_partials(o3, t3)
    tc_partial = _tc_loss_partial(o2, t2)
    return (jnp.sum(sc_partials) + tc_partial) / _TOTAL


# speedup vs baseline: 6.9630x; 1.0118x over previous
import functools

import jax
import jax.numpy as jnp
from jax import lax
from jax.experimental import pallas as pl
from jax.experimental.pallas import tpu as pltpu
from jax.experimental.pallas import tpu_sc as plsc

_TOTAL = 4 * 4096 * 2048
_GRP = 2048
_GR = 8
_COLS = 2048
_NW = 32
_UNROLL = 8

_G_SC = 512
_G_W = _G_SC // _NW

_TC_ROW0 = _G_SC * _GR
_TC_BLOCK = 512
_TC_GRID = (_GRP - _G_SC) * _GR // _TC_BLOCK


def _sc_loss_partials(o3, t3):
    mesh = plsc.VectorSubcoreMesh(core_axis_name="c", subcore_axis_name="s")

    @functools.partial(
        pl.kernel,
        mesh=mesh,
        out_type=jax.ShapeDtypeStruct((_NW, 16), jnp.float32),
        scratch_types=[
            pltpu.VMEM((2, _GR, _COLS), jnp.float32),
            pltpu.VMEM((2, _GR, _COLS), jnp.float32),
            pltpu.VMEM((16,), jnp.float32),
            pltpu.SemaphoreType.DMA,
            pltpu.SemaphoreType.DMA,
            pltpu.SemaphoreType.DMA,
            pltpu.SemaphoreType.DMA,
        ],
    )
    def k(o_hbm, t_hbm, out_hbm, o_buf, t_buf, acc_vm, so0, so1, st0, st1):
        wid = lax.axis_index("s") * 2 + lax.axis_index("c")
        g0 = wid * _G_W
        sems_o = (so0, so1)
        sems_t = (st0, st1)

        def copy_o(k_idx, b):
            return pltpu.make_async_copy(
                o_hbm.at[g0 + k_idx], o_buf.at[b], sems_o[b])

        def copy_t(k_idx, b):
            return pltpu.make_async_copy(
                t_hbm.at[g0 + k_idx], t_buf.at[b], sems_t[b])

        def start(k_idx, b):
            copy_o(k_idx, b).start()
            copy_t(k_idx, b).start()

        def wait(k_idx, b):
            copy_o(k_idx, b).wait()
            copy_t(k_idx, b).wait()

        def chunk_sum(b, accs):
            def rbody(r, a_r):
                orow = o_buf.at[b].at[r]
                trow = t_buf.at[b].at[r]

                def vbody(v, a):
                    out = []
                    for u in range(_UNROLL):
                        off = v * _UNROLL * 16 + u * 16
                        o = orow[pl.ds(off, 16)]
                        t = trow[pl.ds(off, 16)]
                        d = jnp.where(jnp.abs(t) > 0.0, o - t, 0.0)
                        out.append(a[u] + d * d)
                    return tuple(out)

                return lax.fori_loop(0, _COLS // (16 * _UNROLL), vbody, a_r)

            return lax.fori_loop(0, _GR, rbody, accs)

        start(0, 0)
        start(1, 1)

        def gbody(gg, accs):
            for b in (0, 1):
                k_idx = 2 * gg + b
                wait(k_idx, b)
                accs = chunk_sum(b, accs)
                start(k_idx + 2, b)
            return accs

        zero = jnp.zeros((16,), jnp.float32)
        accs = lax.fori_loop(0, (_G_W - 2) // 2, gbody, (zero,) * _UNROLL)
        for b in (0, 1):
            wait(_G_W - 2 + b, b)
            accs = chunk_sum(b, accs)

        acc = accs[0]
        for u in range(1, _UNROLL):
            acc = acc + accs[u]
        acc_vm[...] = acc
        pltpu.sync_copy(acc_vm, out_hbm.at[wid])

    return k(o3, t3)


def _tc_body(o_ref, t_ref, out_ref):
    o = o_ref[...]
    t = t_ref[...]
    d = o - t
    sq = jnp.where(jnp.abs(t) > 0.0, d * d, 0.0)
    part = jnp.sum(sq)

    @pl.when(pl.program_id(0) == 0)
    def _():
        out_ref[0, 0] = 0.0

    out_ref[0, 0] += part


def _tc_loss_partial(o2, t2):
    row_blk0 = _TC_ROW0 // _TC_BLOCK
    total = pl.pallas_call(
        _tc_body,
        grid=(_TC_GRID,),
        in_specs=[
            pl.BlockSpec((_TC_BLOCK, _COLS), lambda i: (row_blk0 + i, 0)),
            pl.BlockSpec((_TC_BLOCK, _COLS), lambda i: (row_blk0 + i, 0)),
        ],
        out_specs=pl.BlockSpec(memory_space=pltpu.SMEM),
        out_shape=jax.ShapeDtypeStruct((1, 1), jnp.float32),
    )(o2, t2)
    return total[0, 0]


def kernel(output, target):
    o3 = output.reshape(_GRP, _GR, _COLS)
    t3 = target.reshape(_GRP, _GR, _COLS)
    o2 = output.reshape(_GRP * _GR, _COLS)
    t2 = target.reshape(_GRP * _GR, _COLS)
    sc_partials = _sc_loss_partials(o3, t3)
    tc_partial = _tc_loss_partial(o2, t2)
    return (jnp.sum(sc_partials) + tc_partial) / _TOTAL


# speedup vs baseline: 6.9956x; 1.0047x over previous
import functools

import jax
import jax.numpy as jnp
from jax import lax
from jax.experimental import pallas as pl
from jax.experimental.pallas import tpu as pltpu
from jax.experimental.pallas import tpu_sc as plsc

_TOTAL = 4 * 4096 * 2048
_GRP = 2048
_GR = 8
_COLS = 2048
_NW = 32
_UNROLL = 8

_G_SC = 64
_G_W = _G_SC // _NW

_TC_ROW0 = _G_SC * _GR
_TC_BLOCK = 512
_TC_GRID = (_GRP - _G_SC) * _GR // _TC_BLOCK


def _sc_loss_partials(o3, t3):
    mesh = plsc.VectorSubcoreMesh(core_axis_name="c", subcore_axis_name="s")

    @functools.partial(
        pl.kernel,
        mesh=mesh,
        out_type=jax.ShapeDtypeStruct((_NW, 16), jnp.float32),
        scratch_types=[
            pltpu.VMEM((2, _GR, _COLS), jnp.float32),
            pltpu.VMEM((2, _GR, _COLS), jnp.float32),
            pltpu.VMEM((16,), jnp.float32),
            pltpu.SemaphoreType.DMA,
            pltpu.SemaphoreType.DMA,
            pltpu.SemaphoreType.DMA,
            pltpu.SemaphoreType.DMA,
        ],
    )
    def k(o_hbm, t_hbm, out_hbm, o_buf, t_buf, acc_vm, so0, so1, st0, st1):
        wid = lax.axis_index("s") * 2 + lax.axis_index("c")
        g0 = wid * _G_W
        sems_o = (so0, so1)
        sems_t = (st0, st1)

        def copy_o(k_idx, b):
            return pltpu.make_async_copy(
                o_hbm.at[g0 + k_idx], o_buf.at[b], sems_o[b])

        def copy_t(k_idx, b):
            return pltpu.make_async_copy(
                t_hbm.at[g0 + k_idx], t_buf.at[b], sems_t[b])

        def start(k_idx, b):
            copy_o(k_idx, b).start()
            copy_t(k_idx, b).start()

        def wait(k_idx, b):
            copy_o(k_idx, b).wait()
            copy_t(k_idx, b).wait()

        def chunk_sum(b, accs):
            def rbody(r, a_r):
                orow = o_buf.at[b].at[r]
                trow = t_buf.at[b].at[r]

                def vbody(v, a):
                    out = []
                    for u in range(_UNROLL):
                        off = v * _UNROLL * 16 + u * 16
                        o = orow[pl.ds(off, 16)]
                        t = trow[pl.ds(off, 16)]
                        d = jnp.where(jnp.abs(t) > 0.0, o - t, 0.0)
                        out.append(a[u] + d * d)
                    return tuple(out)

                return lax.fori_loop(0, _COLS // (16 * _UNROLL), vbody, a_r)

            return lax.fori_loop(0, _GR, rbody, accs)

        start(0, 0)
        start(1, 1)

        def gbody(gg, accs):
            for b in (0, 1):
                k_idx = 2 * gg + b
                wait(k_idx, b)
                accs = chunk_sum(b, accs)
                start(k_idx + 2, b)
            return accs

        zero = jnp.zeros((16,), jnp.float32)
        accs = lax.fori_loop(0, (_G_W - 2) // 2, gbody, (zero,) * _UNROLL)
        for b in (0, 1):
            wait(_G_W - 2 + b, b)
            accs = chunk_sum(b, accs)

        acc = accs[0]
        for u in range(1, _UNROLL):
            acc = acc + accs[u]
        acc_vm[...] = acc
        pltpu.sync_copy(acc_vm, out_hbm.at[wid])

    return k(o3, t3)


def _tc_body(o_ref, t_ref, out_ref):
    o = o_ref[...]
    t = t_ref[...]
    d = o - t
    sq = jnp.where(jnp.abs(t) > 0.0, d * d, 0.0)
    part = jnp.sum(sq)

    @pl.when(pl.program_id(0) == 0)
    def _():
        out_ref[0, 0] = 0.0

    out_ref[0, 0] += part


def _tc_loss_partial(o2, t2):
    row_blk0 = _TC_ROW0 // _TC_BLOCK
    total = pl.pallas_call(
        _tc_body,
        grid=(_TC_GRID,),
        in_specs=[
            pl.BlockSpec((_TC_BLOCK, _COLS), lambda i: (row_blk0 + i, 0)),
            pl.BlockSpec((_TC_BLOCK, _COLS), lambda i: (row_blk0 + i, 0)),
        ],
        out_specs=pl.BlockSpec(memory_space=pltpu.SMEM),
        out_shape=jax.ShapeDtypeStruct((1, 1), jnp.float32),
    )(o2, t2)
    return total[0, 0]


def kernel(output, target):
    o3 = output.reshape(_GRP, _GR, _COLS)
    t3 = target.reshape(_GRP, _GR, _COLS)
    o2 = output.reshape(_GRP * _GR, _COLS)
    t2 = target.reshape(_GRP * _GR, _COLS)
    sc_partials = _sc_loss_partials(o3, t3)
    tc_partial = _tc_loss_partial(o2, t2)
    return (jnp.sum(sc_partials) + tc_partial) / _TOTAL
